# baseline (speedup 1.0000x reference)
import functools

import jax
import jax.numpy as jnp
import numpy as np
from jax import lax
from jax.experimental import pallas as pl
from jax.experimental.pallas import tpu as pltpu
from jax.experimental.pallas import tpu_sc as plsc

_FMAP = 1024

_DROP_IDX = [
    4, 5, 8, 11, 13, 15, 20, 29, 31, 36, 43, 45, 56, 64, 66, 68, 70, 72, 73,
    80, 81, 82, 84, 87, 90, 92, 93, 94, 95, 96, 97, 105, 108, 109, 114, 116,
    121, 131, 137, 140, 143, 148, 149, 154, 162, 168, 172, 188, 189, 192, 196,
    197, 198, 206, 209, 211, 212, 219, 221, 226, 231, 237, 241, 247, 250, 252,
    258, 264, 266, 267, 269, 273, 274, 277, 278, 279, 283, 291, 299, 302, 304,
    305, 319, 321, 322, 325, 326, 327, 331, 333, 338, 342, 343, 347, 348, 354,
    356, 357, 358, 359, 362, 364, 375, 377, 379, 389, 392, 393, 397, 403, 405,
    409, 411, 418, 419, 423, 430, 432, 434, 435, 440, 447, 450, 452, 454, 457,
    471, 472, 473, 474, 478, 479, 482, 491, 503, 510, 511, 512, 516, 517, 519,
    524, 525, 526, 530, 534, 554, 555, 559, 561, 563, 566, 567, 569, 570, 582,
    586, 590, 601, 606, 607, 615, 618, 623, 625, 631, 643, 645, 649, 652, 653,
    654, 657, 666, 671, 675, 679, 691, 694, 695, 701, 703, 707, 710, 711, 713,
    714, 715, 716, 729, 741, 745, 755, 760, 766, 768, 769, 770, 771, 773, 778,
    779, 785, 807, 808, 815, 816, 831, 832, 844, 845, 860, 862, 869, 886, 892,
    902, 903, 905, 906, 909, 911, 924, 931, 935, 939, 940, 945, 946, 948, 953,
    962, 965, 968, 974, 975, 980, 982, 983, 987, 990, 991, 998, 1006, 1010,
    1012, 1013, 1015, 1016, 1017, 1019,
]

_TO_SWAP = np.zeros(_FMAP, dtype=bool)
_TO_SWAP[_DROP_IDX] = True
_KEEP_F32 = (~_TO_SWAP).astype(np.float32)

_ROWS = 4 * 2048
_NC, _NS, _L = 2, 16, 16
_NW = _NC * _NS
_RPW = _ROWS // _NW
_CH = 32
_NCH = _RPW // _CH

_mesh = plsc.VectorSubcoreMesh(core_axis_name="c", subcore_axis_name="s")


@functools.partial(
    pl.kernel,
    mesh=_mesh,
    out_type=jax.ShapeDtypeStruct((_ROWS, _FMAP), jnp.float32),
    scratch_types=[
        pltpu.VMEM((_CH, _FMAP), jnp.float32),
        pltpu.VMEM((_FMAP,), jnp.float32),
        pltpu.VMEM((_FMAP,), jnp.float32),
    ],
)
def _sc_run(x_hbm, embm_hbm, keep_hbm, out_hbm, buf, embm_v, keep_v):
    wid = lax.axis_index("s") * _NC + lax.axis_index("c")
    base = wid * _RPW
    pltpu.sync_copy(embm_hbm, embm_v)
    pltpu.sync_copy(keep_hbm, keep_v)

    def chunk_body(ci, carry):
        row0 = base + ci * _CH
        pltpu.sync_copy(x_hbm.at[pl.ds(row0, _CH)], buf)

        def row_body(r, carry2):
            for j in range(_FMAP // _L):
                s = pl.ds(j * _L, _L)
                buf[r, s] = buf[r, s] * keep_v[s] + embm_v[s]
            return carry2

        lax.fori_loop(0, _CH, row_body, 0)
        pltpu.sync_copy(buf, out_hbm.at[pl.ds(row0, _CH)])
        return carry

    lax.fori_loop(0, _NCH, chunk_body, 0)


def kernel(features, feature_dropout_embed):
    mask = jnp.asarray(_TO_SWAP)
    keep = jnp.asarray(_KEEP_F32)
    emb_masked = jnp.where(mask, feature_dropout_embed, 0.0)
    x2d = features.reshape(_ROWS, _FMAP)
    out = _sc_run(x2d, emb_masked, keep)
    return out.reshape(features.shape)
